# baseline (speedup 1.0000x reference)
import functools

import jax
import jax.numpy as jnp
from jax import lax
from jax.experimental import pallas as pl
from jax.experimental.pallas import tpu as pltpu
from jax.experimental.pallas import tpu_sc as plsc


def _logits_body(gwT_ref, xT_ref, gb_ref, lg_ref):
    lg_ref[...] = (
        jnp.dot(gwT_ref[...], xT_ref[...], preferred_element_type=jnp.float32)
        + gb_ref[...]
    )


def _route_body(lg_hbm, cT_hbm, lg_v, p_v, cT_v):
    cid = lax.axis_index("c")
    sid = lax.axis_index("s")
    E, B = lg_v.shape
    NG = B // 16
    pltpu.sync_copy(lg_hbm, lg_v)
    neg = jnp.zeros((16,), jnp.float32) - 1.0
    zero_i = jnp.zeros((16,), jnp.int32)
    for g in range(NG):
        ds = pl.ds(16 * g, 16)
        s = jnp.zeros((16,), jnp.float32)
        m1 = neg
        m2 = neg
        i1 = zero_i
        i2 = zero_i
        for e in range(E):
            p = jnp.exp(lg_v[e, ds])
            p_v[e, ds] = p
            s = s + p
            ev = zero_i + e
            gt1 = p > m1
            gt2 = jnp.logical_and(jnp.logical_not(gt1), p > m2)
            m2 = jnp.where(gt1, m1, jnp.where(gt2, p, m2))
            i2 = jnp.where(gt1, i1, jnp.where(gt2, ev, i2))
            m1 = jnp.where(gt1, p, m1)
            i1 = jnp.where(gt1, ev, i1)
        inv_s = 1.0 / s
        for e in range(E):
            sel = jnp.logical_or(i1 == e, i2 == e)
            cT_v[e, ds] = jnp.where(sel, p_v[e, ds] * inv_s, 0.0)

    @pl.when((cid == 0) & (sid == 0))
    def _publish():
        pltpu.sync_copy(cT_v, cT_hbm)


def _moe_body(x_ref, cT_ref, w_ref, b_ref, out_ref, *, eb):
    step = pl.program_id(0)
    B = x_ref.shape[0]
    E = cT_ref.shape[0]

    @pl.when(step == 0)
    def _init():
        out_ref[...] = jnp.zeros_like(out_ref)

    acc = out_ref[...]
    ri = lax.broadcasted_iota(jnp.int32, (B, B), 0)
    ci = lax.broadcasted_iota(jnp.int32, (B, B), 1)
    r_iota = lax.broadcasted_iota(jnp.int32, (E, B), 0)
    for j in range(eb):
        e = step * eb + j
        contrib = (
            jnp.dot(x_ref[...], w_ref[j], preferred_element_type=jnp.float32)
            + b_ref[j]
        )
        crow = jnp.sum(jnp.where(r_iota == e, cT_ref[...], 0.0), axis=0, keepdims=True)
        diag = jnp.where(ri == ci, jnp.broadcast_to(crow, (B, B)), 0.0)
        acc = acc + jnp.dot(diag, contrib, preferred_element_type=jnp.float32)
    out_ref[...] = acc


def kernel(x, experts_weights, experts_bias, gate_w, gate_b):
    B, D_in = x.shape
    E, _, D_out = experts_weights.shape

    lg = pl.pallas_call(
        _logits_body,
        out_shape=jax.ShapeDtypeStruct((E, B), jnp.float32),
    )(gate_w.T, x.T, gate_b.reshape(E, 1))

    route = pl.kernel(
        _route_body,
        out_type=jax.ShapeDtypeStruct((E, B), jnp.float32),
        scratch_types=[
            pltpu.VMEM((E, B), jnp.float32),
            pltpu.VMEM((E, B), jnp.float32),
            pltpu.VMEM((E, B), jnp.float32),
        ],
        mesh=plsc.VectorSubcoreMesh(core_axis_name="c", subcore_axis_name="s"),
        compiler_params=pltpu.CompilerParams(needs_layout_passes=False),
    )
    cT = route(lg)

    EB = 4
    out = pl.pallas_call(
        functools.partial(_moe_body, eb=EB),
        grid=(E // EB,),
        in_specs=[
            pl.BlockSpec((B, D_in), lambda e: (0, 0)),
            pl.BlockSpec((E, B), lambda e: (0, 0)),
            pl.BlockSpec((EB, D_in, D_out), lambda e: (e, 0, 0)),
            pl.BlockSpec((EB, 1, D_out), lambda e: (e, 0, 0)),
        ],
        out_specs=pl.BlockSpec((B, D_out), lambda e: (0, 0)),
        out_shape=jax.ShapeDtypeStruct((B, D_out), jnp.float32),
    )(x, cT, experts_weights, experts_bias.reshape(E, 1, D_out))
    return out


# speedup vs baseline: 1.1237x; 1.1237x over previous
import functools

import jax
import jax.numpy as jnp
from jax import lax
from jax.experimental import pallas as pl
from jax.experimental.pallas import tpu as pltpu
from jax.experimental.pallas import tpu_sc as plsc


def _gating_body(gwT_ref, xT_ref, gb_ref, bias_ref, cT_ref, touched_ref, bpart_ref):
    E, B = cT_ref.shape
    logits = (
        jnp.dot(gwT_ref[...], xT_ref[...], preferred_element_type=jnp.float32)
        + gb_ref[...]
    )
    m = jnp.max(logits, axis=0, keepdims=True)
    p = jnp.exp(logits - m)
    g = p / jnp.sum(p, axis=0, keepdims=True)

    r_iota = lax.broadcasted_iota(jnp.int32, (E, B), 0)
    m1 = jnp.max(g, axis=0, keepdims=True)
    idx1 = jnp.min(jnp.where(g == m1, r_iota, E), axis=0, keepdims=True)
    oh1 = r_iota == idx1
    gm = jnp.where(oh1, -1.0, g)
    m2 = jnp.max(gm, axis=0, keepdims=True)
    idx2 = jnp.min(jnp.where(gm == m2, r_iota, E), axis=0, keepdims=True)
    oh2 = r_iota == idx2
    sel = oh1 | oh2
    cT = jnp.where(sel, g, 0.0)
    cT_ref[...] = cT
    touched_ref[...] = jnp.max(sel.astype(jnp.int32), axis=1, keepdims=True)
    bpart_ref[...] = lax.dot_general(
        cT, bias_ref[...], (((0,), (0,)), ((), ())),
        preferred_element_type=jnp.float32,
    )


def _route_body(touched_hbm, ids_hbm, nv_hbm, touched_v, ids_v, nv_v):
    cid = lax.axis_index("c")
    sid = lax.axis_index("s")
    E = touched_hbm.shape[0]
    NCH = E // 16
    pltpu.sync_copy(touched_hbm, touched_v)
    n = jnp.int32(0)
    lastid = jnp.int32(0)
    for k in range(NCH):
        t = touched_v[pl.ds(16 * k, 16)]
        lane = lax.iota(jnp.int32, 16)
        eids = lane + 16 * k
        sel = t > 0
        _, sv = plsc.sort_key_val(jnp.where(sel, lane, lane + 16), eids)
        ids_v[pl.ds(n, 16)] = sv
        n = n + jnp.sum(sel.astype(jnp.int32))
        lastid = jnp.maximum(lastid, jnp.max(jnp.where(sel, eids, 0)))
    for k in range(NCH):
        lane_p = lax.iota(jnp.int32, 16) + 16 * k
        cur = ids_v[pl.ds(16 * k, 16)]
        keep = lane_p < n
        ids_v[pl.ds(16 * k, 16)] = jnp.where(keep, cur, lastid)
    nv_v[pl.ds(0, 16)] = jnp.zeros((16,), jnp.int32) + n

    @pl.when((cid == 0) & (sid == 0))
    def _publish():
        pltpu.sync_copy(ids_v.at[pl.ds(0, E)], ids_hbm)
        pltpu.sync_copy(nv_v, nv_hbm)


_NBUF = 4


def _moe_body(ids_ref, nv_ref, x_ref, cT_ref, bpart_ref, w_hbm, out_ref, wbuf, sems):
    n = nv_ref[0]
    B = x_ref.shape[0]
    E = cT_ref.shape[0]
    out_ref[...] = bpart_ref[...]

    def start(i, slot):
        pltpu.make_async_copy(
            w_hbm.at[pl.ds(ids_ref[i], 1)],
            wbuf.at[pl.ds(slot, 1)],
            sems.at[slot],
        ).start()

    for s in range(_NBUF):

        @pl.when(s < n)
        def _(s=s):
            start(s, s)

    ri = lax.broadcasted_iota(jnp.int32, (B, B), 0)
    ci = lax.broadcasted_iota(jnp.int32, (B, B), 1)
    r_iota = lax.broadcasted_iota(jnp.int32, (E, B), 0)

    def body(i4, carry):
        for k in range(_NBUF):
            i = i4 * _NBUF + k

            @pl.when(i < n)
            def _(i=i, k=k):
                pltpu.make_async_copy(
                    w_hbm.at[pl.ds(ids_ref[i], 1)],
                    wbuf.at[pl.ds(k, 1)],
                    sems.at[k],
                ).wait()
                e_id = ids_ref[i]
                contrib = jnp.dot(
                    x_ref[...], wbuf[k], preferred_element_type=jnp.float32
                )
                crow = jnp.sum(
                    jnp.where(r_iota == e_id, cT_ref[...], 0.0),
                    axis=0,
                    keepdims=True,
                )
                diag = jnp.where(ri == ci, jnp.broadcast_to(crow, (B, B)), 0.0)
                out_ref[...] += jnp.dot(
                    diag, contrib, preferred_element_type=jnp.float32
                )

                @pl.when(i + _NBUF < n)
                def _start_next():
                    start(i + _NBUF, k)

        return carry

    lax.fori_loop(0, (n + _NBUF - 1) // _NBUF, body, 0)


def kernel(x, experts_weights, experts_bias, gate_w, gate_b):
    B, D_in = x.shape
    E, _, D_out = experts_weights.shape

    cT, touched, bpart = pl.pallas_call(
        _gating_body,
        out_shape=[
            jax.ShapeDtypeStruct((E, B), jnp.float32),
            jax.ShapeDtypeStruct((E, 1), jnp.int32),
            jax.ShapeDtypeStruct((B, D_out), jnp.float32),
        ],
    )(gate_w.T, x.T, gate_b.reshape(E, 1), experts_bias)

    route = pl.kernel(
        _route_body,
        out_type=[
            jax.ShapeDtypeStruct((E,), jnp.int32),
            jax.ShapeDtypeStruct((16,), jnp.int32),
        ],
        scratch_types=[
            pltpu.VMEM((E,), jnp.int32),
            pltpu.VMEM((E + 16,), jnp.int32),
            pltpu.VMEM((16,), jnp.int32),
        ],
        mesh=plsc.VectorSubcoreMesh(core_axis_name="c", subcore_axis_name="s"),
        compiler_params=pltpu.CompilerParams(needs_layout_passes=False),
    )
    ids, nv = route(touched.reshape(E))

    out = pl.pallas_call(
        _moe_body,
        in_specs=[
            pl.BlockSpec(memory_space=pltpu.SMEM),
            pl.BlockSpec(memory_space=pltpu.SMEM),
            pl.BlockSpec((B, D_in), lambda: (0, 0)),
            pl.BlockSpec((E, B), lambda: (0, 0)),
            pl.BlockSpec((B, D_out), lambda: (0, 0)),
            pl.BlockSpec(memory_space=pltpu.MemorySpace.HBM),
        ],
        out_specs=pl.BlockSpec((B, D_out), lambda: (0, 0)),
        out_shape=jax.ShapeDtypeStruct((B, D_out), jnp.float32),
        scratch_shapes=[
            pltpu.VMEM((_NBUF, D_in, D_out), jnp.float32),
            pltpu.SemaphoreType.DMA((_NBUF,)),
        ],
    )(ids, nv, x, cT, bpart, experts_weights)
    return out
